# Initial kernel scaffold; baseline (speedup 1.0000x reference)
import jax
import jax.numpy as jnp
from jax.experimental import pallas as pl


def kernel(x, edge_index, edge_attr, W1, b1, W2, b2, W3, b3):
    raise NotImplementedError("write your pallas kernel here")


# speedup vs baseline: 25.1546x; 25.1546x over previous
import functools

import jax
import jax.numpy as jnp
from jax import lax
from jax.experimental import pallas as pl
from jax.experimental.pallas import tpu as pltpu
from jax.experimental.pallas import tpu_sc as plsc

N = 10000
E = 320000
NC, NS = 2, 16
NT = NC * NS
CHUNK = 128
EPT = 10240
NCHUNK = EPT // CHUNK
EPAD = NT * EPT - E
NPAD = 10240
RPT = NPAD // NS
ROWBLK = 400
GRID = N // ROWBLK

_mesh = plsc.VectorSubcoreMesh(
    core_axis_name="c", subcore_axis_name="s", num_cores=NC, num_subcores=NS)


def _make_agg(w):
  @functools.partial(
      pl.kernel,
      out_type=jax.ShapeDtypeStruct((NC, NPAD, w), jnp.float32),
      mesh=_mesh,
      compiler_params=pltpu.CompilerParams(use_tc_tiling_on_sc=False),
      scratch_types=[
          pltpu.VMEM((NCHUNK, CHUNK), jnp.int32),
          pltpu.VMEM((NCHUNK, CHUNK), jnp.int32),
          pltpu.VMEM((CHUNK, w), jnp.float32),
          pltpu.VMEM_SHARED((NPAD, w), jnp.float32),
          pltpu.SemaphoreType.DMA,
      ],
  )
  def agg(g_hbm, src_hbm, dst_hbm, zeros_hbm, out_hbm,
          src_v, dst_v, rows_v, acc, gsem):
    c = lax.axis_index("c")
    s = lax.axis_index("s")
    tile = s * NC + c
    pltpu.sync_copy(zeros_hbm.at[pl.ds(s * RPT, RPT)],
                    acc.at[pl.ds(s * RPT, RPT)])
    pltpu.sync_copy(src_hbm.at[tile], src_v)
    pltpu.sync_copy(dst_hbm.at[tile], dst_v)
    plsc.subcore_barrier()

    def body(j, carry):
      pltpu.async_copy(g_hbm.at[src_v.at[j]], rows_v, gsem).wait()
      pltpu.sync_copy(rows_v, acc.at[dst_v.at[j]], add=True)
      return carry

    lax.fori_loop(0, NCHUNK, body, 0)
    plsc.subcore_barrier()
    pltpu.sync_copy(acc.at[pl.ds(s * RPT, RPT)],
                    out_hbm.at[c, pl.ds(s * RPT, RPT)])

  return agg


_agg32 = _make_agg(32)
_agg16 = _make_agg(16)


@functools.partial(
    pl.kernel,
    out_type=jax.ShapeDtypeStruct((NC, NPAD, 8), jnp.float32),
    mesh=_mesh,
    compiler_params=pltpu.CompilerParams(use_tc_tiling_on_sc=False),
    scratch_types=[
        pltpu.VMEM((NCHUNK, CHUNK), jnp.int32),
        pltpu.VMEM((CHUNK, 8), jnp.float32),
        pltpu.VMEM_SHARED((NPAD, 8), jnp.float32),
    ],
)
def _deg(dst_hbm, ones_hbm, zeros_hbm, out_hbm, dst_v, ones_v, acc):
  c = lax.axis_index("c")
  s = lax.axis_index("s")
  tile = s * NC + c
  pltpu.sync_copy(zeros_hbm.at[pl.ds(s * RPT, RPT)],
                  acc.at[pl.ds(s * RPT, RPT)])
  pltpu.sync_copy(dst_hbm.at[tile], dst_v)
  pltpu.sync_copy(ones_hbm, ones_v)
  plsc.subcore_barrier()

  def body(j, carry):
    pltpu.sync_copy(ones_v, acc.at[dst_v.at[j]], add=True)
    return carry

  lax.fori_loop(0, NCHUNK, body, 0)
  plsc.subcore_barrier()
  pltpu.sync_copy(acc.at[pl.ds(s * RPT, RPT)],
                  out_hbm.at[c, pl.ds(s * RPT, RPT)])


def _dinv_of(degp_ref):
  deg = degp_ref[0, :, 0:1] + degp_ref[1, :, 0:1] + 1.0
  return lax.rsqrt(deg)


def _tc_a(x, degp, W1):
  def body(x_ref, degp_ref, w_ref, g_ref):
    dinv = _dinv_of(degp_ref)
    h = jnp.dot(x_ref[...], w_ref[...], preferred_element_type=jnp.float32)
    g_ref[...] = dinv * h

  return pl.pallas_call(
      body,
      grid=(GRID,),
      in_specs=[
          pl.BlockSpec((ROWBLK, 128), lambda i: (i, 0)),
          pl.BlockSpec((2, ROWBLK, 8), lambda i: (0, i, 0)),
          pl.BlockSpec((128, 32), lambda i: (0, 0)),
      ],
      out_specs=pl.BlockSpec((ROWBLK, 32), lambda i: (i, 0)),
      out_shape=jax.ShapeDtypeStruct((N, 32), jnp.float32),
  )(x, degp, W1)


def _make_tc_mid(w_out):
  def body(p_ref, g_ref, degp_ref, b_ref, w_ref, o_ref):
    dinv = _dinv_of(degp_ref)
    s = dinv * (p_ref[0] + p_ref[1] + g_ref[...]) + b_ref[...]
    o = jnp.maximum(s, 0.0)
    o_ref[...] = dinv * jnp.dot(o, w_ref[...],
                                preferred_element_type=jnp.float32)

  def call(p, g, degp, b, Wn):
    return pl.pallas_call(
        body,
        grid=(GRID,),
        in_specs=[
            pl.BlockSpec((2, ROWBLK, 32), lambda i: (0, i, 0)),
            pl.BlockSpec((ROWBLK, 32), lambda i: (i, 0)),
            pl.BlockSpec((2, ROWBLK, 8), lambda i: (0, i, 0)),
            pl.BlockSpec((1, 32), lambda i: (0, 0)),
            pl.BlockSpec((32, w_out), lambda i: (0, 0)),
        ],
        out_specs=pl.BlockSpec((ROWBLK, w_out), lambda i: (i, 0)),
        out_shape=jax.ShapeDtypeStruct((N, w_out), jnp.float32),
    )(p, g, degp, b, Wn)

  return call


_tc_mid32 = _make_tc_mid(32)
_tc_mid16 = _make_tc_mid(16)


def _tc_out(p, g, degp, b):
  def body(p_ref, g_ref, degp_ref, b_ref, o_ref):
    dinv = _dinv_of(degp_ref)
    s = dinv * (p_ref[0] + p_ref[1] + g_ref[...]) + b_ref[...]
    col = lax.broadcasted_iota(jnp.int32, (ROWBLK, 16), 1)
    mask = col < 12
    m = jnp.max(jnp.where(mask, s, -1e30), axis=1, keepdims=True)
    e = jnp.where(mask, jnp.exp(s - m), 0.0)
    lse = jnp.log(jnp.sum(e, axis=1, keepdims=True))
    o_ref[...] = s - m - lse

  return pl.pallas_call(
      body,
      grid=(GRID,),
      in_specs=[
          pl.BlockSpec((2, ROWBLK, 16), lambda i: (0, i, 0)),
          pl.BlockSpec((ROWBLK, 16), lambda i: (i, 0)),
          pl.BlockSpec((2, ROWBLK, 8), lambda i: (0, i, 0)),
          pl.BlockSpec((1, 16), lambda i: (0, 0)),
      ],
      out_specs=pl.BlockSpec((ROWBLK, 16), lambda i: (i, 0)),
      out_shape=jax.ShapeDtypeStruct((N, 16), jnp.float32),
  )(p, g, degp, b)


def kernel(x, edge_index, edge_attr, W1, b1, W2, b2, W3, b3):
  f32 = jnp.float32
  src = edge_index[0].astype(jnp.int32)
  dst = edge_index[1].astype(jnp.int32)
  ar = jnp.arange(EPAD, dtype=jnp.int32)
  pad_src = (ar * 37) % N
  pad_dst = N + ar % (NPAD - N)
  srcp = jnp.concatenate([src, pad_src]).reshape(NT, NCHUNK, CHUNK)
  dstp = jnp.concatenate([dst, pad_dst]).reshape(NT, NCHUNK, CHUNK)

  zeros8 = jnp.zeros((NPAD, 8), f32)
  zeros32 = jnp.zeros((NPAD, 32), f32)
  zeros16 = jnp.zeros((NPAD, 16), f32)
  ones8 = jnp.ones((CHUNK, 8), f32)
  W3p = jnp.pad(W3, ((0, 0), (0, 4)))
  b3p = jnp.pad(b3, (0, 4)).reshape(1, 16)

  degp = _deg(dstp, ones8, zeros8)[:, :N, :]

  g1 = _tc_a(x, degp, W1)
  p1 = _agg32(g1, srcp, dstp, zeros32)[:, :N, :]
  g2 = _tc_mid32(p1, g1, degp, b1.reshape(1, 32), W2)
  p2 = _agg32(g2, srcp, dstp, zeros32)[:, :N, :]
  g3 = _tc_mid16(p2, g2, degp, b2.reshape(1, 32), W3p)
  p3 = _agg16(g3, srcp, dstp, zeros16)[:, :N, :]
  out = _tc_out(p3, g3, degp, b3p)
  return out[:, :12]
